# Initial kernel scaffold; baseline (speedup 1.0000x reference)
import functools

import jax
import jax.numpy as jnp
from jax import lax
from jax.experimental import pallas as pl
from jax.experimental.pallas import tpu as pltpu
from jax.experimental.pallas import tpu_sc as plsc

N = 10000
E = 160000
D = 256
DE = 16
HALF = N // 2
PADH = 5120
DUMP = 5100
CHUNK = 80
EPT = E // 16
NCHUNK = EPT // CHUNK
TPR = PADH // 16


def _sc_segment_sums(nfeats, srci, dsti, efeats):
    mesh = plsc.VectorSubcoreMesh(core_axis_name="c", subcore_axis_name="s")

    @functools.partial(
        pl.kernel,
        mesh=mesh,
        out_type=[
            jax.ShapeDtypeStruct((2 * PADH, D), jnp.float32),
            jax.ShapeDtypeStruct((2 * PADH, DE), jnp.float32),
            jax.ShapeDtypeStruct((2 * PADH, DE), jnp.float32),
        ],
        scratch_types=[
            pltpu.VMEM((CHUNK,), jnp.int32),
            pltpu.VMEM((CHUNK,), jnp.int32),
            pltpu.VMEM((CHUNK,), jnp.int32),
            pltpu.VMEM((CHUNK, D), jnp.float32),
            pltpu.VMEM((CHUNK, DE), jnp.float32),
            pltpu.VMEM((CHUNK, DE), jnp.float32),
            pltpu.VMEM_SHARED((PADH, D), jnp.float32),
            pltpu.VMEM_SHARED((PADH, DE), jnp.float32),
            pltpu.VMEM_SHARED((PADH, DE), jnp.float32),
            pltpu.SemaphoreType.DMA,
        ],
    )
    def k(nf, srcr, dstr, efr, sn_out, se_out, dg_out,
          src_v, dst_v, loc_v, rows_v, ef_v, ones_v,
          acc_n, acc_e, acc_d, sem):
        c = lax.axis_index("c")
        s = lax.axis_index("s")
        lo = c * HALF

        zeros16 = jnp.zeros((16,), jnp.float32)
        ones16 = jnp.ones((16,), jnp.float32)

        def initrow(j, carry):
            ef_v[j, :] = zeros16
            ones_v[j, :] = ones16
            for t in range(D // 16):
                rows_v[j, pl.ds(t * 16, 16)] = zeros16
            return carry

        lax.fori_loop(0, CHUNK, initrow, 0)

        r0 = s * TPR
        for t in range(TPR // CHUNK):
            pltpu.sync_copy(rows_v, acc_n.at[pl.ds(r0 + t * CHUNK, CHUNK)])
            pltpu.sync_copy(ef_v, acc_e.at[pl.ds(r0 + t * CHUNK, CHUNK)])
            pltpu.sync_copy(ef_v, acc_d.at[pl.ds(r0 + t * CHUNK, CHUNK)])
        plsc.subcore_barrier()

        def chunk_body(i, carry):
            base = s * EPT + i * CHUNK
            pltpu.sync_copy(srcr.at[pl.ds(base, CHUNK)], src_v)
            pltpu.sync_copy(dstr.at[pl.ds(base, CHUNK)], dst_v)
            pltpu.sync_copy(efr.at[pl.ds(base, CHUNK)], ef_v)
            pltpu.async_copy(nf.at[src_v], rows_v, sem).wait()
            for j in range(CHUNK // 16):
                d = dst_v[pl.ds(j * 16, 16)]
                inr = (d >= lo) & (d < lo + HALF)
                loc_v[pl.ds(j * 16, 16)] = jnp.where(inr, d - lo, DUMP)
            pltpu.sync_copy(rows_v, acc_n.at[loc_v], add=True)
            pltpu.sync_copy(ef_v, acc_e.at[loc_v], add=True)
            pltpu.sync_copy(ones_v, acc_d.at[loc_v], add=True)
            return carry

        lax.fori_loop(0, NCHUNK, chunk_body, 0)
        plsc.subcore_barrier()

        o0 = c * PADH + r0
        pltpu.sync_copy(acc_n.at[pl.ds(r0, TPR)], sn_out.at[pl.ds(o0, TPR)])
        pltpu.sync_copy(acc_e.at[pl.ds(r0, TPR)], se_out.at[pl.ds(o0, TPR)])
        pltpu.sync_copy(acc_d.at[pl.ds(r0, TPR)], dg_out.at[pl.ds(o0, TPR)])

    return k(nfeats, srci, dsti, efeats)


def _tc_apply(nfeats, sn, se, dg, w1, w2, wa1, wa2, bm, ba):
    BM = 1000

    def body(nf_ref, sn_ref, se_ref, dg_ref, w1_ref, w2_ref,
             wa1_ref, wa2_ref, bm_ref, ba_ref, out_ref):
        deg = dg_ref[:, 0:1]
        inv = 1.0 / jnp.maximum(deg, 1.0)
        msum = (
            jnp.dot(sn_ref[:, :], w1_ref[:, :],
                    preferred_element_type=jnp.float32,
                    precision=lax.Precision.HIGHEST)
            + jnp.dot(se_ref[:, :], w2_ref[:, :],
                      preferred_element_type=jnp.float32,
                      precision=lax.Precision.HIGHEST)
        )
        h = jnp.where(deg > 0.0, msum * inv + bm_ref[:, :], 0.0)
        acc = (
            jnp.dot(nf_ref[:, :], wa1_ref[:, :],
                    preferred_element_type=jnp.float32,
                    precision=lax.Precision.HIGHEST)
            + jnp.dot(h, wa2_ref[:, :],
                      preferred_element_type=jnp.float32,
                      precision=lax.Precision.HIGHEST)
            + ba_ref[:, :]
        )
        out_ref[:, :] = jnp.maximum(acc, 0.0)

    return pl.pallas_call(
        body,
        grid=(N // BM,),
        in_specs=[
            pl.BlockSpec((BM, D), lambda i: (i, 0)),
            pl.BlockSpec((BM, D), lambda i: (i, 0)),
            pl.BlockSpec((BM, DE), lambda i: (i, 0)),
            pl.BlockSpec((BM, DE), lambda i: (i, 0)),
            pl.BlockSpec((D, D), lambda i: (0, 0)),
            pl.BlockSpec((DE, D), lambda i: (0, 0)),
            pl.BlockSpec((D, D), lambda i: (0, 0)),
            pl.BlockSpec((D, D), lambda i: (0, 0)),
            pl.BlockSpec((1, D), lambda i: (0, 0)),
            pl.BlockSpec((1, D), lambda i: (0, 0)),
        ],
        out_specs=pl.BlockSpec((BM, D), lambda i: (i, 0)),
        out_shape=jax.ShapeDtypeStruct((N, D), jnp.float32),
    )(nfeats, sn, se, dg, w1, w2, wa1, wa2, bm, ba)


def kernel(nfeats, edge_index, efeats, W_msg, b_msg, W_apply, b_apply):
    srci = edge_index[0]
    dsti = edge_index[1]
    sn_p, se_p, dg_p = _sc_segment_sums(nfeats, srci, dsti, efeats)
    sn = jnp.concatenate([sn_p[:HALF], sn_p[PADH:PADH + HALF]], axis=0)
    se = jnp.concatenate([se_p[:HALF], se_p[PADH:PADH + HALF]], axis=0)
    dg = jnp.concatenate([dg_p[:HALF], dg_p[PADH:PADH + HALF]], axis=0)
    w1 = W_msg[:D]
    w2 = W_msg[D:]
    wa1 = W_apply[:D]
    wa2 = W_apply[D:]
    return _tc_apply(nfeats, sn, se, dg, w1, w2, wa1, wa2,
                     b_msg.reshape(1, D), b_apply.reshape(1, D))


# speedup vs baseline: 2.2377x; 2.2377x over previous
import functools

import jax
import jax.numpy as jnp
from jax import lax
from jax.experimental import pallas as pl
from jax.experimental.pallas import tpu as pltpu
from jax.experimental.pallas import tpu_sc as plsc

N = 10000
E = 160000
D = 256
DE = 16
HALF = N // 2
PADH = 5120
DUMP = 5100
CHUNK = 80
EPT = E // 16
NCHUNK = EPT // CHUNK
TPR = PADH // 16


def _sc_segment_sums(nfeats, srci, dsti, efeats):
    mesh = plsc.VectorSubcoreMesh(core_axis_name="c", subcore_axis_name="s")

    @functools.partial(
        pl.kernel,
        mesh=mesh,
        out_type=[
            jax.ShapeDtypeStruct((2 * PADH, D), jnp.float32),
            jax.ShapeDtypeStruct((2 * PADH, 2 * DE), jnp.float32),
        ],
        scratch_types=[
            pltpu.VMEM((CHUNK,), jnp.int32),
            pltpu.VMEM((CHUNK,), jnp.int32),
            pltpu.VMEM((CHUNK,), jnp.int32),
            pltpu.VMEM((CHUNK, D), jnp.float32),
            pltpu.VMEM((CHUNK, 2 * DE), jnp.float32),
            pltpu.VMEM_SHARED((PADH, D), jnp.float32),
            pltpu.VMEM_SHARED((PADH, 2 * DE), jnp.float32),
            pltpu.SemaphoreType.DMA,
        ],
        compiler_params=pltpu.CompilerParams(use_tc_tiling_on_sc=False),
    )
    def k(nf, srcr, dstr, efr, sn_out, sd_out,
          src_v, dst_v, loc_v, rows_v, em_v,
          acc_n, acc_m, sem):
        c = lax.axis_index("c")
        s = lax.axis_index("s")
        lo = c * HALF

        zeros16 = jnp.zeros((16,), jnp.float32)
        ones16 = jnp.ones((16,), jnp.float32)

        def initrow(j, carry):
            em_v[j, pl.ds(0, 16)] = zeros16
            em_v[j, pl.ds(16, 16)] = zeros16
            for t in range(D // 16):
                rows_v[j, pl.ds(t * 16, 16)] = zeros16
            return carry

        lax.fori_loop(0, CHUNK, initrow, 0)

        r0 = s * TPR
        for t in range(TPR // CHUNK):
            pltpu.sync_copy(rows_v, acc_n.at[pl.ds(r0 + t * CHUNK, CHUNK)])
            pltpu.sync_copy(em_v, acc_m.at[pl.ds(r0 + t * CHUNK, CHUNK)])

        def onesrow(j, carry):
            em_v[j, pl.ds(16, 16)] = ones16
            return carry

        lax.fori_loop(0, CHUNK, onesrow, 0)
        plsc.subcore_barrier()

        def chunk_body(i, carry):
            base = s * EPT + i * CHUNK
            pltpu.sync_copy(srcr.at[pl.ds(base, CHUNK)], src_v)
            pltpu.sync_copy(dstr.at[pl.ds(base, CHUNK)], dst_v)
            pltpu.sync_copy(efr.at[pl.ds(base, CHUNK)],
                            em_v.at[pl.ds(0, CHUNK), pl.ds(0, DE)])
            pltpu.async_copy(nf.at[src_v], rows_v, sem).wait()
            for j in range(CHUNK // 16):
                d = dst_v[pl.ds(j * 16, 16)]
                inr = (d >= lo) & (d < lo + HALF)
                loc_v[pl.ds(j * 16, 16)] = jnp.where(inr, d - lo, DUMP)
            pltpu.sync_copy(rows_v, acc_n.at[loc_v], add=True)
            pltpu.sync_copy(em_v, acc_m.at[loc_v], add=True)
            return carry

        lax.fori_loop(0, NCHUNK, chunk_body, 0)
        plsc.subcore_barrier()

        o0 = c * PADH + r0
        pltpu.sync_copy(acc_n.at[pl.ds(r0, TPR)], sn_out.at[pl.ds(o0, TPR)])
        pltpu.sync_copy(acc_m.at[pl.ds(r0, TPR)], sd_out.at[pl.ds(o0, TPR)])

    return k(nfeats, srci, dsti, efeats)


def _tc_apply(nfeats, sn, sd, w1, w2, wa1, wa2, bm, ba):
    BM = 1000

    def body(nf_ref, sn_ref, sd_ref, w1_ref, w2_ref,
             wa1_ref, wa2_ref, bm_ref, ba_ref, out_ref):
        deg = sd_ref[:, DE:DE + 1]
        se = sd_ref[:, 0:DE]
        inv = 1.0 / jnp.maximum(deg, 1.0)
        msum = (
            jnp.dot(sn_ref[:, :], w1_ref[:, :],
                    preferred_element_type=jnp.float32,
                    precision=lax.Precision.HIGHEST)
            + jnp.dot(se, w2_ref[:, :],
                      preferred_element_type=jnp.float32,
                      precision=lax.Precision.HIGHEST)
        )
        h = jnp.where(deg > 0.0, msum * inv + bm_ref[:, :], 0.0)
        acc = (
            jnp.dot(nf_ref[:, :], wa1_ref[:, :],
                    preferred_element_type=jnp.float32,
                    precision=lax.Precision.HIGHEST)
            + jnp.dot(h, wa2_ref[:, :],
                      preferred_element_type=jnp.float32,
                      precision=lax.Precision.HIGHEST)
            + ba_ref[:, :]
        )
        out_ref[:, :] = jnp.maximum(acc, 0.0)

    return pl.pallas_call(
        body,
        grid=(N // BM,),
        in_specs=[
            pl.BlockSpec((BM, D), lambda i: (i, 0)),
            pl.BlockSpec((BM, D), lambda i: (i, 0)),
            pl.BlockSpec((BM, 2 * DE), lambda i: (i, 0)),
            pl.BlockSpec((D, D), lambda i: (0, 0)),
            pl.BlockSpec((DE, D), lambda i: (0, 0)),
            pl.BlockSpec((D, D), lambda i: (0, 0)),
            pl.BlockSpec((D, D), lambda i: (0, 0)),
            pl.BlockSpec((1, D), lambda i: (0, 0)),
            pl.BlockSpec((1, D), lambda i: (0, 0)),
        ],
        out_specs=pl.BlockSpec((BM, D), lambda i: (i, 0)),
        out_shape=jax.ShapeDtypeStruct((N, D), jnp.float32),
    )(nfeats, sn, sd, w1, w2, wa1, wa2, bm, ba)


def kernel(nfeats, edge_index, efeats, W_msg, b_msg, W_apply, b_apply):
    srci = edge_index[0]
    dsti = edge_index[1]
    sn_p, sd_p = _sc_segment_sums(nfeats, srci, dsti, efeats)
    sn = jnp.concatenate([sn_p[:HALF], sn_p[PADH:PADH + HALF]], axis=0)
    sd = jnp.concatenate([sd_p[:HALF], sd_p[PADH:PADH + HALF]], axis=0)
    w1 = W_msg[:D]
    w2 = W_msg[D:]
    wa1 = W_apply[:D]
    wa2 = W_apply[D:]
    return _tc_apply(nfeats, sn, sd, w1, w2, wa1, wa2,
                     b_msg.reshape(1, D), b_apply.reshape(1, D))


# speedup vs baseline: 2.4472x; 1.0936x over previous
import functools

import jax
import jax.numpy as jnp
from jax import lax
from jax.experimental import pallas as pl
from jax.experimental.pallas import tpu as pltpu
from jax.experimental.pallas import tpu_sc as plsc

N = 10000
E = 160000
D = 256
DE = 16
HALF = N // 2
PADH = 5120
DUMP = 5100
CHUNK = 80
EPT = E // 16
NCHUNK = EPT // CHUNK
TPR = PADH // 16


def _sc_segment_sums(nfeats, srci, dsti, efeats):
    mesh = plsc.VectorSubcoreMesh(core_axis_name="c", subcore_axis_name="s")

    @functools.partial(
        pl.kernel,
        mesh=mesh,
        out_type=[
            jax.ShapeDtypeStruct((N, D), jnp.float32),
            jax.ShapeDtypeStruct((N, 2 * DE), jnp.float32),
        ],
        scratch_types=[
            pltpu.VMEM((CHUNK,), jnp.int32),
            pltpu.VMEM((CHUNK,), jnp.int32),
            pltpu.VMEM((CHUNK,), jnp.int32),
            pltpu.VMEM((CHUNK, D), jnp.float32),
            pltpu.VMEM((CHUNK, 2 * DE), jnp.float32),
            pltpu.VMEM_SHARED((PADH, D), jnp.float32),
            pltpu.VMEM_SHARED((PADH, 2 * DE), jnp.float32),
            pltpu.SemaphoreType.DMA,
        ],
        compiler_params=pltpu.CompilerParams(use_tc_tiling_on_sc=False),
    )
    def k(nf, srcr, dstr, efr, sn_out, sd_out,
          src_v, dst_v, loc_v, rows_v, em_v,
          acc_n, acc_m, sem):
        c = lax.axis_index("c")
        s = lax.axis_index("s")
        lo = c * HALF

        zeros16 = jnp.zeros((16,), jnp.float32)
        ones16 = jnp.ones((16,), jnp.float32)

        def initrow(j, carry):
            em_v[j, pl.ds(0, 16)] = zeros16
            em_v[j, pl.ds(16, 16)] = zeros16
            for t in range(D // 16):
                rows_v[j, pl.ds(t * 16, 16)] = zeros16
            return carry

        lax.fori_loop(0, CHUNK, initrow, 0)

        r0 = s * TPR
        for t in range(TPR // CHUNK):
            pltpu.sync_copy(rows_v, acc_n.at[pl.ds(r0 + t * CHUNK, CHUNK)])
            pltpu.sync_copy(em_v, acc_m.at[pl.ds(r0 + t * CHUNK, CHUNK)])

        def onesrow(j, carry):
            em_v[j, pl.ds(16, 16)] = ones16
            return carry

        lax.fori_loop(0, CHUNK, onesrow, 0)
        plsc.subcore_barrier()

        def chunk_body(i, carry):
            base = s * EPT + i * CHUNK
            pltpu.sync_copy(srcr.at[pl.ds(base, CHUNK)], src_v)
            pltpu.sync_copy(dstr.at[pl.ds(base, CHUNK)], dst_v)
            pltpu.sync_copy(efr.at[pl.ds(base, CHUNK)],
                            em_v.at[pl.ds(0, CHUNK), pl.ds(0, DE)])
            pltpu.async_copy(nf.at[src_v], rows_v, sem).wait()
            for j in range(CHUNK // 16):
                d = dst_v[pl.ds(j * 16, 16)]
                inr = (d >= lo) & (d < lo + HALF)
                loc_v[pl.ds(j * 16, 16)] = jnp.where(inr, d - lo, DUMP)
            pltpu.sync_copy(rows_v, acc_n.at[loc_v], add=True)
            pltpu.sync_copy(em_v, acc_m.at[loc_v], add=True)
            return carry

        lax.fori_loop(0, NCHUNK, chunk_body, 0)
        plsc.subcore_barrier()

        o0 = c * HALF + r0
        pltpu.sync_copy(acc_n.at[pl.ds(r0, 200)], sn_out.at[pl.ds(o0, 200)])
        pltpu.sync_copy(acc_m.at[pl.ds(r0, 200)], sd_out.at[pl.ds(o0, 200)])

        @pl.when(s < 15)
        def _():
            pltpu.sync_copy(acc_n.at[pl.ds(r0 + 200, 120)],
                            sn_out.at[pl.ds(o0 + 200, 120)])
            pltpu.sync_copy(acc_m.at[pl.ds(r0 + 200, 120)],
                            sd_out.at[pl.ds(o0 + 200, 120)])

    return k(nfeats, srci, dsti, efeats)


_F32 = jnp.float32
_HI = lax.Precision.HIGHEST


def _tc_pre(nfeats, wa1, w1, w2, wa2, bm, ba):
    BM = 1000

    def body(nf_ref, wa1_ref, w1_ref, w2_ref, wa2_ref, bm_ref, ba_ref,
             p_ref, v1_ref, v2_ref, vb_ref):
        p_ref[:, :] = (
            jnp.dot(nf_ref[:, :], wa1_ref[:, :],
                    preferred_element_type=_F32, precision=_HI)
            + ba_ref[:, :]
        )
        @pl.when(pl.program_id(0) == 0)
        def _():
            v1_ref[:, :] = jnp.dot(w1_ref[:, :], wa2_ref[:, :],
                                   preferred_element_type=_F32, precision=_HI)
            v2_ref[:, :] = jnp.dot(w2_ref[:, :], wa2_ref[:, :],
                                   preferred_element_type=_F32, precision=_HI)
            vb_ref[:, :] = jnp.dot(bm_ref[:, :], wa2_ref[:, :],
                                   preferred_element_type=_F32, precision=_HI)

    return pl.pallas_call(
        body,
        grid=(N // BM,),
        in_specs=[
            pl.BlockSpec((BM, D), lambda i: (i, 0)),
            pl.BlockSpec((D, D), lambda i: (0, 0)),
            pl.BlockSpec((D, D), lambda i: (0, 0)),
            pl.BlockSpec((DE, D), lambda i: (0, 0)),
            pl.BlockSpec((D, D), lambda i: (0, 0)),
            pl.BlockSpec((1, D), lambda i: (0, 0)),
            pl.BlockSpec((1, D), lambda i: (0, 0)),
        ],
        out_specs=[
            pl.BlockSpec((BM, D), lambda i: (i, 0)),
            pl.BlockSpec((D, D), lambda i: (0, 0)),
            pl.BlockSpec((DE, D), lambda i: (0, 0)),
            pl.BlockSpec((1, D), lambda i: (0, 0)),
        ],
        out_shape=[
            jax.ShapeDtypeStruct((N, D), _F32),
            jax.ShapeDtypeStruct((D, D), _F32),
            jax.ShapeDtypeStruct((DE, D), _F32),
            jax.ShapeDtypeStruct((1, D), _F32),
        ],
    )(nfeats, wa1, w1, w2, wa2, bm, ba)


def _tc_post(p, sn_p, sd_p, v1, v2, vb):
    BM = 1000

    def body(p_ref, sn_ref, sd_ref, v1_ref, v2_ref, vb_ref, out_ref):
        deg = sd_ref[:, DE:DE + 1]
        inv = jnp.where(deg > 0.0, 1.0 / jnp.maximum(deg, 1.0), 0.0)
        msk = jnp.where(deg > 0.0, 1.0, 0.0)
        acc = (
            p_ref[:, :]
            + jnp.dot(sn_ref[:, :] * inv, v1_ref[:, :],
                      preferred_element_type=_F32, precision=_HI)
            + jnp.dot(sd_ref[:, 0:DE] * inv, v2_ref[:, :],
                      preferred_element_type=_F32, precision=_HI)
            + msk * vb_ref[:, :]
        )
        out_ref[:, :] = jnp.maximum(acc, 0.0)

    return pl.pallas_call(
        body,
        grid=(N // BM,),
        in_specs=[
            pl.BlockSpec((BM, D), lambda i: (i, 0)),
            pl.BlockSpec((BM, D), lambda i: (i, 0)),
            pl.BlockSpec((BM, 2 * DE), lambda i: (i, 0)),
            pl.BlockSpec((D, D), lambda i: (0, 0)),
            pl.BlockSpec((DE, D), lambda i: (0, 0)),
            pl.BlockSpec((1, D), lambda i: (0, 0)),
        ],
        out_specs=pl.BlockSpec((BM, D), lambda i: (i, 0)),
        out_shape=jax.ShapeDtypeStruct((N, D), _F32),
    )(p, sn_p, sd_p, v1, v2, vb)


def kernel(nfeats, edge_index, efeats, W_msg, b_msg, W_apply, b_apply):
    srci = edge_index[0]
    dsti = edge_index[1]
    w1 = W_msg[:D]
    w2 = W_msg[D:]
    wa1 = W_apply[:D]
    wa2 = W_apply[D:]
    p, v1, v2, vb = _tc_pre(nfeats, wa1, w1, w2, wa2,
                            b_msg.reshape(1, D), b_apply.reshape(1, D))
    sn_p, sd_p = _sc_segment_sums(nfeats, srci, dsti, efeats)
    return _tc_post(p, sn_p, sd_p, v1, v2, vb)


# speedup vs baseline: 2.8863x; 1.1794x over previous
import functools

import jax
import jax.numpy as jnp
from jax import lax
from jax.experimental import pallas as pl
from jax.experimental.pallas import tpu as pltpu
from jax.experimental.pallas import tpu_sc as plsc

N = 10000
E = 160000
D = 256
DE = 16
HALF = N // 2
PADH = 5120
DUMP = 5100
CHUNK = 80
EPT = E // 16
NCHUNK = EPT // CHUNK
TPR = PADH // 16


def _sc_segment_sums(nfeats, srci, dsti, efeats):
    mesh = plsc.VectorSubcoreMesh(core_axis_name="c", subcore_axis_name="s")

    @functools.partial(
        pl.kernel,
        mesh=mesh,
        out_type=[
            jax.ShapeDtypeStruct((N, D), jnp.float32),
            jax.ShapeDtypeStruct((N, 2 * DE), jnp.bfloat16),
        ],
        scratch_types=[
            pltpu.VMEM((2 * CHUNK,), jnp.int32),
            pltpu.VMEM((2 * CHUNK,), jnp.int32),
            pltpu.VMEM((CHUNK,), jnp.int32),
            pltpu.VMEM((CHUNK,), jnp.int32),
            pltpu.VMEM((2 * CHUNK, D), jnp.float32),
            pltpu.VMEM((CHUNK, 2 * DE), jnp.bfloat16),
            pltpu.VMEM_SHARED((PADH, D), jnp.float32),
            pltpu.VMEM_SHARED((PADH, 2 * DE), jnp.bfloat16),
            pltpu.SemaphoreType.DMA,
            pltpu.SemaphoreType.DMA,
        ],
        compiler_params=pltpu.CompilerParams(use_tc_tiling_on_sc=False),
    )
    def k(nf, srcr, dstr, efr, sn_out, sd_out,
          src_v, dst_v, loc0, loc1, rows_v, em_v,
          acc_n, acc_m, sem0, sem1):
        c = lax.axis_index("c")
        s = lax.axis_index("s")
        lo = c * HALF

        zeros16 = jnp.zeros((16,), jnp.float32)
        zeros32b = jnp.zeros((32,), jnp.bfloat16)

        def initrow(j, carry):
            for t in range(D // 16):
                rows_v[j, pl.ds(t * 16, 16)] = zeros16
            return carry

        lax.fori_loop(0, 2 * CHUNK, initrow, 0)

        def initem(j, carry):
            em_v[j, :] = zeros32b
            return carry

        lax.fori_loop(0, CHUNK, initem, 0)

        r0 = s * TPR
        for t in range(TPR // (2 * CHUNK)):
            pltpu.sync_copy(rows_v,
                            acc_n.at[pl.ds(r0 + t * 2 * CHUNK, 2 * CHUNK)])
        for t in range(TPR // CHUNK):
            pltpu.sync_copy(em_v, acc_m.at[pl.ds(r0 + t * CHUNK, CHUNK)])
        plsc.subcore_barrier()

        def fire(i, h, sem):
            base = s * EPT + i * CHUNK
            pltpu.sync_copy(srcr.at[pl.ds(base, CHUNK)],
                            src_v.at[pl.ds(h * CHUNK, CHUNK)])
            pltpu.sync_copy(dstr.at[pl.ds(base, CHUNK)],
                            dst_v.at[pl.ds(h * CHUNK, CHUNK)])
            return pltpu.async_copy(
                nf.at[src_v.at[pl.ds(h * CHUNK, CHUNK)]],
                rows_v.at[pl.ds(h * CHUNK, CHUNK)], sem)

        def process(i, h, locb):
            base = s * EPT + i * CHUNK
            pltpu.sync_copy(efr.at[pl.ds(base, CHUNK)], em_v)
            for j in range(CHUNK // 16):
                d = dst_v[pl.ds(h * CHUNK + j * 16, 16)]
                inr = (d >= lo) & (d < lo + HALF)
                locb[pl.ds(j * 16, 16)] = jnp.where(inr, d - lo, DUMP)
            pltpu.sync_copy(rows_v.at[pl.ds(h * CHUNK, CHUNK)],
                            acc_n.at[locb], add=True)
            pltpu.sync_copy(em_v, acc_m.at[locb], add=True)

        def pair_body(p, carry):
            g0 = fire(2 * p, 0, sem0)
            g1 = fire(2 * p + 1, 1, sem1)
            g0.wait()
            process(2 * p, 0, loc0)
            g1.wait()
            process(2 * p + 1, 1, loc1)
            return carry

        lax.fori_loop(0, NCHUNK // 2, pair_body, 0)
        if NCHUNK % 2:
            fire(NCHUNK - 1, 0, sem0).wait()
            process(NCHUNK - 1, 0, loc0)
        plsc.subcore_barrier()

        o0 = c * HALF + r0
        pltpu.sync_copy(acc_n.at[pl.ds(r0, 200)], sn_out.at[pl.ds(o0, 200)])
        pltpu.sync_copy(acc_m.at[pl.ds(r0, 200)], sd_out.at[pl.ds(o0, 200)])

        @pl.when(s < 15)
        def _():
            pltpu.sync_copy(acc_n.at[pl.ds(r0 + 200, 120)],
                            sn_out.at[pl.ds(o0 + 200, 120)])
            pltpu.sync_copy(acc_m.at[pl.ds(r0 + 200, 120)],
                            sd_out.at[pl.ds(o0 + 200, 120)])

    return k(nfeats, srci, dsti, efeats)


_F32 = jnp.float32
_HI = lax.Precision.HIGHEST


def _tc_pre(nfeats, wa1, w1, w2, wa2, bm, ba):
    BM = 1000

    def body(nf_ref, wa1_ref, w1_ref, w2_ref, wa2_ref, bm_ref, ba_ref,
             p_ref, v1_ref, v2_ref, vb_ref):
        p_ref[:, :] = (
            jnp.dot(nf_ref[:, :], wa1_ref[:, :],
                    preferred_element_type=_F32, precision=_HI)
            + ba_ref[:, :]
        )
        @pl.when(pl.program_id(0) == 0)
        def _():
            v1_ref[:, :] = jnp.dot(w1_ref[:, :], wa2_ref[:, :],
                                   preferred_element_type=_F32, precision=_HI)
            v2_ref[:, :] = jnp.dot(w2_ref[:, :], wa2_ref[:, :],
                                   preferred_element_type=_F32, precision=_HI)
            vb_ref[:, :] = jnp.dot(bm_ref[:, :], wa2_ref[:, :],
                                   preferred_element_type=_F32, precision=_HI)

    return pl.pallas_call(
        body,
        grid=(N // BM,),
        in_specs=[
            pl.BlockSpec((BM, D), lambda i: (i, 0)),
            pl.BlockSpec((D, D), lambda i: (0, 0)),
            pl.BlockSpec((D, D), lambda i: (0, 0)),
            pl.BlockSpec((DE, D), lambda i: (0, 0)),
            pl.BlockSpec((D, D), lambda i: (0, 0)),
            pl.BlockSpec((1, D), lambda i: (0, 0)),
            pl.BlockSpec((1, D), lambda i: (0, 0)),
        ],
        out_specs=[
            pl.BlockSpec((BM, D), lambda i: (i, 0)),
            pl.BlockSpec((D, D), lambda i: (0, 0)),
            pl.BlockSpec((DE, D), lambda i: (0, 0)),
            pl.BlockSpec((1, D), lambda i: (0, 0)),
        ],
        out_shape=[
            jax.ShapeDtypeStruct((N, D), _F32),
            jax.ShapeDtypeStruct((D, D), _F32),
            jax.ShapeDtypeStruct((DE, D), _F32),
            jax.ShapeDtypeStruct((1, D), _F32),
        ],
    )(nfeats, wa1, w1, w2, wa2, bm, ba)


def _tc_post(p, sn_p, sd_p, v1, v2, vb):
    BM = 1000

    def body(p_ref, sn_ref, sd_ref, v1_ref, v2_ref, vb_ref, out_ref):
        sd = sd_ref[:, :].astype(_F32)
        deg = sd[:, DE:DE + 1]
        inv = jnp.where(deg > 0.0, 1.0 / jnp.maximum(deg, 1.0), 0.0)
        msk = jnp.where(deg > 0.0, 1.0, 0.0)
        acc = (
            p_ref[:, :]
            + jnp.dot(sn_ref[:, :] * inv, v1_ref[:, :],
                      preferred_element_type=_F32, precision=_HI)
            + jnp.dot(sd[:, 0:DE] * inv, v2_ref[:, :],
                      preferred_element_type=_F32, precision=_HI)
            + msk * vb_ref[:, :]
        )
        out_ref[:, :] = jnp.maximum(acc, 0.0)

    return pl.pallas_call(
        body,
        grid=(N // BM,),
        in_specs=[
            pl.BlockSpec((BM, D), lambda i: (i, 0)),
            pl.BlockSpec((BM, D), lambda i: (i, 0)),
            pl.BlockSpec((BM, 2 * DE), lambda i: (i, 0)),
            pl.BlockSpec((D, D), lambda i: (0, 0)),
            pl.BlockSpec((DE, D), lambda i: (0, 0)),
            pl.BlockSpec((1, D), lambda i: (0, 0)),
        ],
        out_specs=pl.BlockSpec((BM, D), lambda i: (i, 0)),
        out_shape=jax.ShapeDtypeStruct((N, D), _F32),
    )(p, sn_p, sd_p, v1, v2, vb)


def kernel(nfeats, edge_index, efeats, W_msg, b_msg, W_apply, b_apply):
    srci = edge_index[0]
    dsti = edge_index[1]
    ef_aug = jnp.concatenate(
        [efeats.astype(jnp.bfloat16),
         jnp.ones((E, DE), jnp.bfloat16)], axis=1)
    w1 = W_msg[:D]
    w2 = W_msg[D:]
    wa1 = W_apply[:D]
    wa2 = W_apply[D:]
    p, v1, v2, vb = _tc_pre(nfeats, wa1, w1, w2, wa2,
                            b_msg.reshape(1, D), b_apply.reshape(1, D))
    sn_p, sd_p = _sc_segment_sums(nfeats, srci, dsti, ef_aug)
    return _tc_post(p, sn_p, sd_p, v1, v2, vb)


# speedup vs baseline: 3.0992x; 1.0738x over previous
import functools

import jax
import jax.numpy as jnp
from jax import lax
from jax.experimental import pallas as pl
from jax.experimental.pallas import tpu as pltpu
from jax.experimental.pallas import tpu_sc as plsc

N = 10000
E = 160000
D = 256
DE = 16
HALF = N // 2
PADH = 5120
DUMP = 5100
CHUNK = 80
EPT = E // 16
NCHUNK = EPT // CHUNK
TPR = PADH // 16


def _sc_segment_sums(nfeats, srci, dsti, efeats):
    mesh = plsc.VectorSubcoreMesh(core_axis_name="c", subcore_axis_name="s")

    @functools.partial(
        pl.kernel,
        mesh=mesh,
        out_type=[
            jax.ShapeDtypeStruct((N, D), jnp.float32),
            jax.ShapeDtypeStruct((N, 2 * DE), jnp.bfloat16),
        ],
        scratch_types=[
            pltpu.VMEM((2 * CHUNK,), jnp.int32),
            pltpu.VMEM((2 * CHUNK,), jnp.int32),
            pltpu.VMEM((CHUNK,), jnp.int32),
            pltpu.VMEM((CHUNK,), jnp.int32),
            pltpu.VMEM((2 * CHUNK, D), jnp.float32),
            pltpu.VMEM((CHUNK, 2 * DE), jnp.bfloat16),
            pltpu.VMEM_SHARED((PADH, D), jnp.float32),
            pltpu.VMEM_SHARED((PADH, 2 * DE), jnp.bfloat16),
            pltpu.SemaphoreType.DMA,
            pltpu.SemaphoreType.DMA,
        ],
        compiler_params=pltpu.CompilerParams(use_tc_tiling_on_sc=False),
    )
    def k(nf, srcr, dstr, efr, sn_out, sd_out,
          src_v, dst_v, loc0, loc1, rows_v, em_v,
          acc_n, acc_m, sem0, sem1):
        c = lax.axis_index("c")
        s = lax.axis_index("s")
        lo = c * HALF

        zeros16 = jnp.zeros((16,), jnp.float32)
        zeros32b = jnp.zeros((32,), jnp.bfloat16)

        def initrow(j, carry):
            for t in range(D // 16):
                rows_v[j, pl.ds(t * 16, 16)] = zeros16
            return carry

        lax.fori_loop(0, 2 * CHUNK, initrow, 0)

        def initem(j, carry):
            em_v[j, :] = zeros32b
            return carry

        lax.fori_loop(0, CHUNK, initem, 0)

        r0 = s * TPR
        for t in range(TPR // (2 * CHUNK)):
            pltpu.sync_copy(rows_v,
                            acc_n.at[pl.ds(r0 + t * 2 * CHUNK, 2 * CHUNK)])
        for t in range(TPR // CHUNK):
            pltpu.sync_copy(em_v, acc_m.at[pl.ds(r0 + t * CHUNK, CHUNK)])
        plsc.subcore_barrier()

        def fire(i, h, sem):
            base = s * EPT + i * CHUNK
            pltpu.sync_copy(srcr.at[pl.ds(base, CHUNK)],
                            src_v.at[pl.ds(h * CHUNK, CHUNK)])
            pltpu.sync_copy(dstr.at[pl.ds(base, CHUNK)],
                            dst_v.at[pl.ds(h * CHUNK, CHUNK)])
            return pltpu.async_copy(
                nf.at[src_v.at[pl.ds(h * CHUNK, CHUNK)]],
                rows_v.at[pl.ds(h * CHUNK, CHUNK)], sem)

        def process(i, h, locb):
            base = s * EPT + i * CHUNK
            pltpu.sync_copy(efr.at[pl.ds(base, CHUNK)], em_v)
            for j in range(CHUNK // 16):
                d = dst_v[pl.ds(h * CHUNK + j * 16, 16)]
                inr = (d >= lo) & (d < lo + HALF)
                locb[pl.ds(j * 16, 16)] = jnp.where(inr, d - lo, DUMP)
            pltpu.sync_copy(rows_v.at[pl.ds(h * CHUNK, CHUNK)],
                            acc_n.at[locb], add=True)
            pltpu.sync_copy(em_v, acc_m.at[locb], add=True)

        def gwait(h, sem):
            pltpu.make_async_copy(
                nf.at[src_v.at[pl.ds(h * CHUNK, CHUNK)]],
                rows_v.at[pl.ds(h * CHUNK, CHUNK)], sem).wait()

        fire(0, 0, sem0)

        def pair_body(p, carry):
            fire(2 * p + 1, 1, sem1)
            gwait(0, sem0)
            process(2 * p, 0, loc0)
            fire(2 * p + 2, 0, sem0)
            gwait(1, sem1)
            process(2 * p + 1, 1, loc1)
            return carry

        lax.fori_loop(0, (NCHUNK - 1) // 2, pair_body, 0)
        gwait(0, sem0)
        process(NCHUNK - 1, 0, loc0)
        plsc.subcore_barrier()

        o0 = c * HALF + r0
        pltpu.sync_copy(acc_n.at[pl.ds(r0, 200)], sn_out.at[pl.ds(o0, 200)])
        pltpu.sync_copy(acc_m.at[pl.ds(r0, 200)], sd_out.at[pl.ds(o0, 200)])

        @pl.when(s < 15)
        def _():
            pltpu.sync_copy(acc_n.at[pl.ds(r0 + 200, 120)],
                            sn_out.at[pl.ds(o0 + 200, 120)])
            pltpu.sync_copy(acc_m.at[pl.ds(r0 + 200, 120)],
                            sd_out.at[pl.ds(o0 + 200, 120)])

    return k(nfeats, srci, dsti, efeats)


_F32 = jnp.float32
_HI = lax.Precision.HIGHEST


def _tc_pre(nfeats, wa1, w1, w2, wa2, bm, ba):
    BM = 1000

    def body(nf_ref, wa1_ref, w1_ref, w2_ref, wa2_ref, bm_ref, ba_ref,
             p_ref, v1_ref, v2_ref, vb_ref):
        p_ref[:, :] = (
            jnp.dot(nf_ref[:, :], wa1_ref[:, :],
                    preferred_element_type=_F32, precision=_HI)
            + ba_ref[:, :]
        )
        @pl.when(pl.program_id(0) == 0)
        def _():
            v1_ref[:, :] = jnp.dot(w1_ref[:, :], wa2_ref[:, :],
                                   preferred_element_type=_F32, precision=_HI)
            v2_ref[:, :] = jnp.dot(w2_ref[:, :], wa2_ref[:, :],
                                   preferred_element_type=_F32, precision=_HI)
            vb_ref[:, :] = jnp.dot(bm_ref[:, :], wa2_ref[:, :],
                                   preferred_element_type=_F32, precision=_HI)

    return pl.pallas_call(
        body,
        grid=(N // BM,),
        in_specs=[
            pl.BlockSpec((BM, D), lambda i: (i, 0)),
            pl.BlockSpec((D, D), lambda i: (0, 0)),
            pl.BlockSpec((D, D), lambda i: (0, 0)),
            pl.BlockSpec((DE, D), lambda i: (0, 0)),
            pl.BlockSpec((D, D), lambda i: (0, 0)),
            pl.BlockSpec((1, D), lambda i: (0, 0)),
            pl.BlockSpec((1, D), lambda i: (0, 0)),
        ],
        out_specs=[
            pl.BlockSpec((BM, D), lambda i: (i, 0)),
            pl.BlockSpec((D, D), lambda i: (0, 0)),
            pl.BlockSpec((DE, D), lambda i: (0, 0)),
            pl.BlockSpec((1, D), lambda i: (0, 0)),
        ],
        out_shape=[
            jax.ShapeDtypeStruct((N, D), _F32),
            jax.ShapeDtypeStruct((D, D), _F32),
            jax.ShapeDtypeStruct((DE, D), _F32),
            jax.ShapeDtypeStruct((1, D), _F32),
        ],
    )(nfeats, wa1, w1, w2, wa2, bm, ba)


def _tc_post(p, sn_p, sd_p, v1, v2, vb):
    BM = 1000

    def body(p_ref, sn_ref, sd_ref, v1_ref, v2_ref, vb_ref, out_ref):
        sd = sd_ref[:, :].astype(_F32)
        deg = sd[:, DE:DE + 1]
        inv = jnp.where(deg > 0.0, 1.0 / jnp.maximum(deg, 1.0), 0.0)
        msk = jnp.where(deg > 0.0, 1.0, 0.0)
        acc = (
            p_ref[:, :]
            + jnp.dot(sn_ref[:, :] * inv, v1_ref[:, :],
                      preferred_element_type=_F32, precision=_HI)
            + jnp.dot(sd[:, 0:DE] * inv, v2_ref[:, :],
                      preferred_element_type=_F32, precision=_HI)
            + msk * vb_ref[:, :]
        )
        out_ref[:, :] = jnp.maximum(acc, 0.0)

    return pl.pallas_call(
        body,
        grid=(N // BM,),
        in_specs=[
            pl.BlockSpec((BM, D), lambda i: (i, 0)),
            pl.BlockSpec((BM, D), lambda i: (i, 0)),
            pl.BlockSpec((BM, 2 * DE), lambda i: (i, 0)),
            pl.BlockSpec((D, D), lambda i: (0, 0)),
            pl.BlockSpec((DE, D), lambda i: (0, 0)),
            pl.BlockSpec((1, D), lambda i: (0, 0)),
        ],
        out_specs=pl.BlockSpec((BM, D), lambda i: (i, 0)),
        out_shape=jax.ShapeDtypeStruct((N, D), _F32),
    )(p, sn_p, sd_p, v1, v2, vb)


def kernel(nfeats, edge_index, efeats, W_msg, b_msg, W_apply, b_apply):
    srci = edge_index[0]
    dsti = edge_index[1]
    ef_aug = jnp.concatenate(
        [efeats.astype(jnp.bfloat16),
         jnp.ones((E, DE), jnp.bfloat16)], axis=1)
    w1 = W_msg[:D]
    w2 = W_msg[D:]
    wa1 = W_apply[:D]
    wa2 = W_apply[D:]
    p, v1, v2, vb = _tc_pre(nfeats, wa1, w1, w2, wa2,
                            b_msg.reshape(1, D), b_apply.reshape(1, D))
    sn_p, sd_p = _sc_segment_sums(nfeats, srci, dsti, ef_aug)
    return _tc_post(p, sn_p, sd_p, v1, v2, vb)


# speedup vs baseline: 3.3122x; 1.0687x over previous
import functools

import jax
import jax.numpy as jnp
from jax import lax
from jax.experimental import pallas as pl
from jax.experimental.pallas import tpu as pltpu
from jax.experimental.pallas import tpu_sc as plsc

N = 10000
E = 160000
D = 256
DE = 16
HALF = N // 2
PADH = 5120
DUMP = 5100
CHUNK = 80
EPT = E // 16
NCHUNK = EPT // CHUNK
TPR = PADH // 16


def _sc_segment_sums(nfeats, sd_pack, ef_aug):
    mesh = plsc.VectorSubcoreMesh(core_axis_name="c", subcore_axis_name="s")

    @functools.partial(
        pl.kernel,
        mesh=mesh,
        out_type=[
            jax.ShapeDtypeStruct((N, D), jnp.float32),
            jax.ShapeDtypeStruct((N, 2 * DE), jnp.bfloat16),
        ],
        scratch_types=[
            pltpu.VMEM((2 * CHUNK,), jnp.int32),
            pltpu.VMEM((2 * CHUNK,), jnp.int32),
            pltpu.VMEM((CHUNK,), jnp.int32),
            pltpu.VMEM((CHUNK,), jnp.int32),
            pltpu.VMEM((2 * CHUNK, D), jnp.float32),
            pltpu.VMEM((CHUNK, 2 * DE), jnp.bfloat16),
            pltpu.VMEM_SHARED((PADH, D), jnp.float32),
            pltpu.VMEM_SHARED((PADH, 2 * DE), jnp.bfloat16),
            pltpu.SemaphoreType.DMA,
            pltpu.SemaphoreType.DMA,
        ],
        compiler_params=pltpu.CompilerParams(use_tc_tiling_on_sc=False),
    )
    def k(nf, sdr, efr, sn_out, sd_out,
          sd0, sd1, loc0, loc1, rows_v, em_v,
          acc_n, acc_m, sem0, sem1):
        c = lax.axis_index("c")
        s = lax.axis_index("s")
        lo = c * HALF

        zeros16 = jnp.zeros((16,), jnp.float32)
        zeros32b = jnp.zeros((32,), jnp.bfloat16)

        def initrow(j, carry):
            for t in range(D // 16):
                rows_v[j, pl.ds(t * 16, 16)] = zeros16
            return carry

        lax.fori_loop(0, 2 * CHUNK, initrow, 0)

        def initem(j, carry):
            em_v[j, :] = zeros32b
            return carry

        lax.fori_loop(0, CHUNK, initem, 0)

        r0 = s * TPR
        for t in range(TPR // (2 * CHUNK)):
            pltpu.sync_copy(rows_v,
                            acc_n.at[pl.ds(r0 + t * 2 * CHUNK, 2 * CHUNK)])
        for t in range(TPR // CHUNK):
            pltpu.sync_copy(em_v, acc_m.at[pl.ds(r0 + t * CHUNK, CHUNK)])
        plsc.subcore_barrier()

        sdb = (sd0, sd1)

        def fire(i, h, sem):
            g = s * NCHUNK + i
            pltpu.sync_copy(sdr.at[pl.ds(g * 2 * CHUNK, 2 * CHUNK)], sdb[h])
            return pltpu.async_copy(
                nf.at[sdb[h].at[pl.ds(0, CHUNK)]],
                rows_v.at[pl.ds(h * CHUNK, CHUNK)], sem)

        def process(i, h, locb):
            base = s * EPT + i * CHUNK
            pltpu.sync_copy(efr.at[pl.ds(base, CHUNK)], em_v)
            for j in range(CHUNK // 16):
                d = sdb[h][pl.ds(CHUNK + j * 16, 16)]
                inr = (d >= lo) & (d < lo + HALF)
                locb[pl.ds(j * 16, 16)] = jnp.where(inr, d - lo, DUMP)
            pltpu.sync_copy(rows_v.at[pl.ds(h * CHUNK, CHUNK)],
                            acc_n.at[locb], add=True)
            pltpu.sync_copy(em_v, acc_m.at[locb], add=True)

        def gwait(h, sem):
            pltpu.make_async_copy(
                nf.at[sdb[h].at[pl.ds(0, CHUNK)]],
                rows_v.at[pl.ds(h * CHUNK, CHUNK)], sem).wait()

        fire(0, 0, sem0)

        def pair_body(p, carry):
            fire(2 * p + 1, 1, sem1)
            gwait(0, sem0)
            process(2 * p, 0, loc0)
            fire(2 * p + 2, 0, sem0)
            gwait(1, sem1)
            process(2 * p + 1, 1, loc1)
            return carry

        lax.fori_loop(0, (NCHUNK - 1) // 2, pair_body, 0)
        gwait(0, sem0)
        process(NCHUNK - 1, 0, loc0)
        plsc.subcore_barrier()

        o0 = c * HALF + r0
        pltpu.sync_copy(acc_n.at[pl.ds(r0, 200)], sn_out.at[pl.ds(o0, 200)])
        pltpu.sync_copy(acc_m.at[pl.ds(r0, 200)], sd_out.at[pl.ds(o0, 200)])

        @pl.when(s < 15)
        def _():
            pltpu.sync_copy(acc_n.at[pl.ds(r0 + 200, 120)],
                            sn_out.at[pl.ds(o0 + 200, 120)])
            pltpu.sync_copy(acc_m.at[pl.ds(r0 + 200, 120)],
                            sd_out.at[pl.ds(o0 + 200, 120)])

    return k(nfeats, sd_pack, ef_aug)


_F32 = jnp.float32
_HI = lax.Precision.HIGHEST


def _tc_pre(nfeats, wa1, w1, w2, wa2, bm, ba):
    BM = 1000

    def body(nf_ref, wa1_ref, w1_ref, w2_ref, wa2_ref, bm_ref, ba_ref,
             p_ref, v1_ref, v2_ref, vb_ref):
        p_ref[:, :] = (
            jnp.dot(nf_ref[:, :], wa1_ref[:, :],
                    preferred_element_type=_F32, precision=_HI)
            + ba_ref[:, :]
        )
        @pl.when(pl.program_id(0) == 0)
        def _():
            v1_ref[:, :] = jnp.dot(w1_ref[:, :], wa2_ref[:, :],
                                   preferred_element_type=_F32, precision=_HI)
            v2_ref[:, :] = jnp.dot(w2_ref[:, :], wa2_ref[:, :],
                                   preferred_element_type=_F32, precision=_HI)
            vb_ref[:, :] = jnp.dot(bm_ref[:, :], wa2_ref[:, :],
                                   preferred_element_type=_F32, precision=_HI)

    return pl.pallas_call(
        body,
        grid=(N // BM,),
        in_specs=[
            pl.BlockSpec((BM, D), lambda i: (i, 0)),
            pl.BlockSpec((D, D), lambda i: (0, 0)),
            pl.BlockSpec((D, D), lambda i: (0, 0)),
            pl.BlockSpec((DE, D), lambda i: (0, 0)),
            pl.BlockSpec((D, D), lambda i: (0, 0)),
            pl.BlockSpec((1, D), lambda i: (0, 0)),
            pl.BlockSpec((1, D), lambda i: (0, 0)),
        ],
        out_specs=[
            pl.BlockSpec((BM, D), lambda i: (i, 0)),
            pl.BlockSpec((D, D), lambda i: (0, 0)),
            pl.BlockSpec((DE, D), lambda i: (0, 0)),
            pl.BlockSpec((1, D), lambda i: (0, 0)),
        ],
        out_shape=[
            jax.ShapeDtypeStruct((N, D), _F32),
            jax.ShapeDtypeStruct((D, D), _F32),
            jax.ShapeDtypeStruct((DE, D), _F32),
            jax.ShapeDtypeStruct((1, D), _F32),
        ],
    )(nfeats, wa1, w1, w2, wa2, bm, ba)


def _tc_post(p, sn_p, sd_p, v1, v2, vb):
    BM = 1000

    def body(p_ref, sn_ref, sd_ref, v1_ref, v2_ref, vb_ref, out_ref):
        sd = sd_ref[:, :].astype(_F32)
        deg = sd[:, DE:DE + 1]
        inv = jnp.where(deg > 0.0, 1.0 / jnp.maximum(deg, 1.0), 0.0)
        msk = jnp.where(deg > 0.0, 1.0, 0.0)
        acc = (
            p_ref[:, :]
            + jnp.dot(sn_ref[:, :] * inv, v1_ref[:, :],
                      preferred_element_type=_F32, precision=_HI)
            + jnp.dot(sd[:, 0:DE] * inv, v2_ref[:, :],
                      preferred_element_type=_F32, precision=_HI)
            + msk * vb_ref[:, :]
        )
        out_ref[:, :] = jnp.maximum(acc, 0.0)

    return pl.pallas_call(
        body,
        grid=(N // BM,),
        in_specs=[
            pl.BlockSpec((BM, D), lambda i: (i, 0)),
            pl.BlockSpec((BM, D), lambda i: (i, 0)),
            pl.BlockSpec((BM, 2 * DE), lambda i: (i, 0)),
            pl.BlockSpec((D, D), lambda i: (0, 0)),
            pl.BlockSpec((DE, D), lambda i: (0, 0)),
            pl.BlockSpec((1, D), lambda i: (0, 0)),
        ],
        out_specs=pl.BlockSpec((BM, D), lambda i: (i, 0)),
        out_shape=jax.ShapeDtypeStruct((N, D), _F32),
    )(p, sn_p, sd_p, v1, v2, vb)


def kernel(nfeats, edge_index, efeats, W_msg, b_msg, W_apply, b_apply):
    srci = edge_index[0]
    dsti = edge_index[1]
    sd_pack = jnp.stack(
        [srci.reshape(E // CHUNK, CHUNK),
         dsti.reshape(E // CHUNK, CHUNK)], axis=1).reshape(-1)
    ef_aug = jnp.concatenate(
        [efeats.astype(jnp.bfloat16),
         jnp.ones((E, DE), jnp.bfloat16)], axis=1)
    w1 = W_msg[:D]
    w2 = W_msg[D:]
    wa1 = W_apply[:D]
    wa2 = W_apply[D:]
    p, v1, v2, vb = _tc_pre(nfeats, wa1, w1, w2, wa2,
                            b_msg.reshape(1, D), b_apply.reshape(1, D))
    sn_p, sd_p = _sc_segment_sums(nfeats, sd_pack, ef_aug)
    return _tc_post(p, sn_p, sd_p, v1, v2, vb)


# speedup vs baseline: 3.6414x; 1.0994x over previous
import functools

import jax
import jax.numpy as jnp
from jax import lax
from jax.experimental import pallas as pl
from jax.experimental.pallas import tpu as pltpu
from jax.experimental.pallas import tpu_sc as plsc

N = 10000
E = 160000
D = 256
DE = 16
HALF = N // 2
PADH = 5120
DUMP = 5100
CHUNK = 80
EPT = E // 16
NCHUNK = EPT // CHUNK
TPR = PADH // 16


def _sc_segment_sums(nfeats, sd_pack, ef_aug):
    mesh = plsc.VectorSubcoreMesh(core_axis_name="c", subcore_axis_name="s")

    @functools.partial(
        pl.kernel,
        mesh=mesh,
        out_type=[
            jax.ShapeDtypeStruct((N, D), jnp.float32),
            jax.ShapeDtypeStruct((N, 2 * DE), jnp.bfloat16),
        ],
        scratch_types=[
            pltpu.VMEM((2 * CHUNK,), jnp.int32),
            pltpu.VMEM((2 * CHUNK,), jnp.int32),
            pltpu.VMEM((CHUNK,), jnp.int32),
            pltpu.VMEM((CHUNK,), jnp.int32),
            pltpu.VMEM((2 * CHUNK, D), jnp.float32),
            pltpu.VMEM((CHUNK, 2 * DE), jnp.bfloat16),
            pltpu.VMEM((CHUNK, 2 * DE), jnp.bfloat16),
            pltpu.VMEM_SHARED((PADH, D), jnp.float32),
            pltpu.VMEM_SHARED((PADH, 2 * DE), jnp.bfloat16),
            pltpu.SemaphoreType.DMA,
            pltpu.SemaphoreType.DMA,
            pltpu.SemaphoreType.DMA,
            pltpu.SemaphoreType.DMA,
        ],
        compiler_params=pltpu.CompilerParams(use_tc_tiling_on_sc=False),
    )
    def k(nf, sdr, efr, sn_out, sd_out,
          sd0, sd1, loc0, loc1, rows_v, em0, em1,
          acc_n, acc_m, sem0, sem1, sem_e0, sem_e1):
        c = lax.axis_index("c")
        s = lax.axis_index("s")
        lo = c * HALF

        zeros16 = jnp.zeros((16,), jnp.float32)
        zeros32b = jnp.zeros((32,), jnp.bfloat16)

        def initrow(j, carry):
            for t in range(D // 16):
                rows_v[j, pl.ds(t * 16, 16)] = zeros16
            return carry

        lax.fori_loop(0, 2 * CHUNK, initrow, 0)

        def initem(j, carry):
            em0[j, :] = zeros32b
            return carry

        lax.fori_loop(0, CHUNK, initem, 0)

        r0 = s * TPR
        for t in range(TPR // (2 * CHUNK)):
            pltpu.sync_copy(rows_v,
                            acc_n.at[pl.ds(r0 + t * 2 * CHUNK, 2 * CHUNK)])
        for t in range(TPR // CHUNK):
            pltpu.sync_copy(em0, acc_m.at[pl.ds(r0 + t * CHUNK, CHUNK)])
        plsc.subcore_barrier()

        sdb = (sd0, sd1)
        emb = (em0, em1)
        esem = (sem_e0, sem_e1)

        def fire(i, h, sem):
            g = s * NCHUNK + i
            pltpu.sync_copy(sdr.at[pl.ds(g * 2 * CHUNK, 2 * CHUNK)], sdb[h])
            pltpu.async_copy(efr.at[pl.ds(s * EPT + i * CHUNK, CHUNK)],
                             emb[h], esem[h])
            return pltpu.async_copy(
                nf.at[sdb[h].at[pl.ds(0, CHUNK)]],
                rows_v.at[pl.ds(h * CHUNK, CHUNK)], sem)

        def process(i, h, locb):
            pltpu.make_async_copy(
                efr.at[pl.ds(0, CHUNK)], emb[h], esem[h]).wait()
            for j in range(CHUNK // 16):
                d = sdb[h][pl.ds(CHUNK + j * 16, 16)]
                inr = (d >= lo) & (d < lo + HALF)
                locb[pl.ds(j * 16, 16)] = jnp.where(inr, d - lo, DUMP)
            pltpu.sync_copy(rows_v.at[pl.ds(h * CHUNK, CHUNK)],
                            acc_n.at[locb], add=True)
            pltpu.sync_copy(emb[h], acc_m.at[locb], add=True)

        def gwait(h, sem):
            pltpu.make_async_copy(
                nf.at[sdb[h].at[pl.ds(0, CHUNK)]],
                rows_v.at[pl.ds(h * CHUNK, CHUNK)], sem).wait()

        fire(0, 0, sem0)

        def pair_body(p, carry):
            fire(2 * p + 1, 1, sem1)
            gwait(0, sem0)
            process(2 * p, 0, loc0)
            fire(2 * p + 2, 0, sem0)
            gwait(1, sem1)
            process(2 * p + 1, 1, loc1)
            return carry

        lax.fori_loop(0, (NCHUNK - 1) // 2, pair_body, 0)
        gwait(0, sem0)
        process(NCHUNK - 1, 0, loc0)
        plsc.subcore_barrier()

        o0 = c * HALF + r0
        pltpu.sync_copy(acc_n.at[pl.ds(r0, 200)], sn_out.at[pl.ds(o0, 200)])
        pltpu.sync_copy(acc_m.at[pl.ds(r0, 200)], sd_out.at[pl.ds(o0, 200)])

        @pl.when(s < 15)
        def _():
            pltpu.sync_copy(acc_n.at[pl.ds(r0 + 200, 120)],
                            sn_out.at[pl.ds(o0 + 200, 120)])
            pltpu.sync_copy(acc_m.at[pl.ds(r0 + 200, 120)],
                            sd_out.at[pl.ds(o0 + 200, 120)])

    return k(nfeats, sd_pack, ef_aug)


_F32 = jnp.float32
_HI = lax.Precision.HIGHEST


def _tc_pre(nfeats, wa1, w1, w2, wa2, bm, ba):
    BM = 1000

    def body(nf_ref, wa1_ref, w1_ref, w2_ref, wa2_ref, bm_ref, ba_ref,
             p_ref, v1_ref, v2_ref, vb_ref):
        p_ref[:, :] = (
            jnp.dot(nf_ref[:, :], wa1_ref[:, :],
                    preferred_element_type=_F32, precision=_HI)
            + ba_ref[:, :]
        )
        @pl.when(pl.program_id(0) == 0)
        def _():
            v1_ref[:, :] = jnp.dot(w1_ref[:, :], wa2_ref[:, :],
                                   preferred_element_type=_F32, precision=_HI)
            v2_ref[:, :] = jnp.dot(w2_ref[:, :], wa2_ref[:, :],
                                   preferred_element_type=_F32, precision=_HI)
            vb_ref[:, :] = jnp.dot(bm_ref[:, :], wa2_ref[:, :],
                                   preferred_element_type=_F32, precision=_HI)

    return pl.pallas_call(
        body,
        grid=(N // BM,),
        in_specs=[
            pl.BlockSpec((BM, D), lambda i: (i, 0)),
            pl.BlockSpec((D, D), lambda i: (0, 0)),
            pl.BlockSpec((D, D), lambda i: (0, 0)),
            pl.BlockSpec((DE, D), lambda i: (0, 0)),
            pl.BlockSpec((D, D), lambda i: (0, 0)),
            pl.BlockSpec((1, D), lambda i: (0, 0)),
            pl.BlockSpec((1, D), lambda i: (0, 0)),
        ],
        out_specs=[
            pl.BlockSpec((BM, D), lambda i: (i, 0)),
            pl.BlockSpec((D, D), lambda i: (0, 0)),
            pl.BlockSpec((DE, D), lambda i: (0, 0)),
            pl.BlockSpec((1, D), lambda i: (0, 0)),
        ],
        out_shape=[
            jax.ShapeDtypeStruct((N, D), _F32),
            jax.ShapeDtypeStruct((D, D), _F32),
            jax.ShapeDtypeStruct((DE, D), _F32),
            jax.ShapeDtypeStruct((1, D), _F32),
        ],
    )(nfeats, wa1, w1, w2, wa2, bm, ba)


def _tc_post(p, sn_p, sd_p, v1, v2, vb):
    BM = 1000

    def body(p_ref, sn_ref, sd_ref, v1_ref, v2_ref, vb_ref, out_ref):
        sd = sd_ref[:, :].astype(_F32)
        deg = sd[:, DE:DE + 1]
        inv = jnp.where(deg > 0.0, 1.0 / jnp.maximum(deg, 1.0), 0.0)
        msk = jnp.where(deg > 0.0, 1.0, 0.0)
        acc = (
            p_ref[:, :]
            + jnp.dot(sn_ref[:, :] * inv, v1_ref[:, :],
                      preferred_element_type=_F32, precision=_HI)
            + jnp.dot(sd[:, 0:DE] * inv, v2_ref[:, :],
                      preferred_element_type=_F32, precision=_HI)
            + msk * vb_ref[:, :]
        )
        out_ref[:, :] = jnp.maximum(acc, 0.0)

    return pl.pallas_call(
        body,
        grid=(N // BM,),
        in_specs=[
            pl.BlockSpec((BM, D), lambda i: (i, 0)),
            pl.BlockSpec((BM, D), lambda i: (i, 0)),
            pl.BlockSpec((BM, 2 * DE), lambda i: (i, 0)),
            pl.BlockSpec((D, D), lambda i: (0, 0)),
            pl.BlockSpec((DE, D), lambda i: (0, 0)),
            pl.BlockSpec((1, D), lambda i: (0, 0)),
        ],
        out_specs=pl.BlockSpec((BM, D), lambda i: (i, 0)),
        out_shape=jax.ShapeDtypeStruct((N, D), _F32),
    )(p, sn_p, sd_p, v1, v2, vb)


def kernel(nfeats, edge_index, efeats, W_msg, b_msg, W_apply, b_apply):
    srci = edge_index[0]
    dsti = edge_index[1]
    sd_pack = jnp.stack(
        [srci.reshape(E // CHUNK, CHUNK),
         dsti.reshape(E // CHUNK, CHUNK)], axis=1).reshape(-1)
    ef_aug = jnp.concatenate(
        [efeats.astype(jnp.bfloat16),
         jnp.ones((E, DE), jnp.bfloat16)], axis=1)
    w1 = W_msg[:D]
    w2 = W_msg[D:]
    wa1 = W_apply[:D]
    wa2 = W_apply[D:]
    p, v1, v2, vb = _tc_pre(nfeats, wa1, w1, w2, wa2,
                            b_msg.reshape(1, D), b_apply.reshape(1, D))
    sn_p, sd_p = _sc_segment_sums(nfeats, sd_pack, ef_aug)
    return _tc_post(p, sn_p, sd_p, v1, v2, vb)


# speedup vs baseline: 3.7100x; 1.0189x over previous
import functools

import jax
import jax.numpy as jnp
from jax import lax
from jax.experimental import pallas as pl
from jax.experimental.pallas import tpu as pltpu
from jax.experimental.pallas import tpu_sc as plsc

N = 10000
E = 160000
D = 256
DE = 16
HALF = N // 2
PADH = 5120
DUMP = 5100
CHUNK = 80
EPT = E // 16
NCHUNK = EPT // CHUNK
TPR = PADH // 16


def _sc_segment_sums(nfeats, sd_pack, ef_aug):
    mesh = plsc.VectorSubcoreMesh(core_axis_name="c", subcore_axis_name="s")

    @functools.partial(
        pl.kernel,
        mesh=mesh,
        out_type=[
            jax.ShapeDtypeStruct((N, D), jnp.float32),
            jax.ShapeDtypeStruct((N, 2 * DE), jnp.bfloat16),
        ],
        scratch_types=[
            pltpu.VMEM((2 * CHUNK,), jnp.int32),
            pltpu.VMEM((2 * CHUNK,), jnp.int32),
            pltpu.VMEM((CHUNK,), jnp.int32),
            pltpu.VMEM((CHUNK,), jnp.int32),
            pltpu.VMEM((2 * CHUNK, D), jnp.float32),
            pltpu.VMEM((CHUNK, 2 * DE), jnp.bfloat16),
            pltpu.VMEM((CHUNK, 2 * DE), jnp.bfloat16),
            pltpu.VMEM_SHARED((PADH, D), jnp.float32),
            pltpu.VMEM_SHARED((PADH, 2 * DE), jnp.bfloat16),
            pltpu.SemaphoreType.DMA,
            pltpu.SemaphoreType.DMA,
            pltpu.SemaphoreType.DMA,
            pltpu.SemaphoreType.DMA,
        ],
        compiler_params=pltpu.CompilerParams(use_tc_tiling_on_sc=False),
    )
    def k(nf, sdr, efr, sn_out, sd_out,
          sd0, sd1, loc0, loc1, rows_v, em0, em1,
          acc_n, acc_m, sem0, sem1, sem_e0, sem_e1):
        c = lax.axis_index("c")
        s = lax.axis_index("s")
        lo = c * HALF

        zeros16 = jnp.zeros((16,), jnp.float32)
        zeros32b = jnp.zeros((32,), jnp.bfloat16)

        def initrow(j, carry):
            for t in range(D // 16):
                rows_v[j, pl.ds(t * 16, 16)] = zeros16
            return carry

        lax.fori_loop(0, 2 * CHUNK, initrow, 0)

        def initem(j, carry):
            em0[j, :] = zeros32b
            return carry

        lax.fori_loop(0, CHUNK, initem, 0)

        r0 = s * TPR
        for t in range(TPR // (2 * CHUNK)):
            pltpu.sync_copy(rows_v,
                            acc_n.at[pl.ds(r0 + t * 2 * CHUNK, 2 * CHUNK)])
        for t in range(TPR // CHUNK):
            pltpu.sync_copy(em0, acc_m.at[pl.ds(r0 + t * CHUNK, CHUNK)])
        plsc.subcore_barrier()

        sdb = (sd0, sd1)
        emb = (em0, em1)
        esem = (sem_e0, sem_e1)

        def fire(i, h, sem):
            g = s * NCHUNK + i
            pltpu.sync_copy(sdr.at[pl.ds(g * 2 * CHUNK, 2 * CHUNK)], sdb[h])
            pltpu.async_copy(efr.at[pl.ds(s * EPT + i * CHUNK, CHUNK)],
                             emb[h], esem[h])
            return pltpu.async_copy(
                nf.at[sdb[h].at[pl.ds(0, CHUNK)]],
                rows_v.at[pl.ds(h * CHUNK, CHUNK)], sem)

        def process(i, h, locb):
            pltpu.make_async_copy(
                efr.at[pl.ds(0, CHUNK)], emb[h], esem[h]).wait()
            for j in range(CHUNK // 16):
                d = sdb[h][pl.ds(CHUNK + j * 16, 16)]
                inr = (d >= lo) & (d < lo + HALF)
                locb[pl.ds(j * 16, 16)] = jnp.where(inr, d - lo, DUMP)
            pltpu.sync_copy(rows_v.at[pl.ds(h * CHUNK, CHUNK)],
                            acc_n.at[locb], add=True)
            pltpu.sync_copy(emb[h], acc_m.at[locb], add=True)

        def gwait(h, sem):
            pltpu.make_async_copy(
                nf.at[sdb[h].at[pl.ds(0, CHUNK)]],
                rows_v.at[pl.ds(h * CHUNK, CHUNK)], sem).wait()

        fire(0, 0, sem0)

        def pair_body(p, carry):
            fire(2 * p + 1, 1, sem1)
            gwait(0, sem0)
            process(2 * p, 0, loc0)
            fire(2 * p + 2, 0, sem0)
            gwait(1, sem1)
            process(2 * p + 1, 1, loc1)
            return carry

        lax.fori_loop(0, (NCHUNK - 1) // 2, pair_body, 0)
        gwait(0, sem0)
        process(NCHUNK - 1, 0, loc0)
        plsc.subcore_barrier()

        o0 = c * HALF + r0
        pltpu.sync_copy(acc_n.at[pl.ds(r0, 200)], sn_out.at[pl.ds(o0, 200)])
        pltpu.sync_copy(acc_m.at[pl.ds(r0, 200)], sd_out.at[pl.ds(o0, 200)])

        @pl.when(s < 15)
        def _():
            pltpu.sync_copy(acc_n.at[pl.ds(r0 + 200, 120)],
                            sn_out.at[pl.ds(o0 + 200, 120)])
            pltpu.sync_copy(acc_m.at[pl.ds(r0 + 200, 120)],
                            sd_out.at[pl.ds(o0 + 200, 120)])

    return k(nfeats, sd_pack, ef_aug)


_F32 = jnp.float32
_HI = lax.Precision.DEFAULT


def _tc_pre(nfeats, wa1, w1, w2, wa2, bm, ba):
    BM = 1000

    def body(nf_ref, wa1_ref, w1_ref, w2_ref, wa2_ref, bm_ref, ba_ref,
             p_ref, v1_ref, v2_ref, vb_ref):
        p_ref[:, :] = (
            jnp.dot(nf_ref[:, :], wa1_ref[:, :],
                    preferred_element_type=_F32, precision=_HI)
            + ba_ref[:, :]
        )
        @pl.when(pl.program_id(0) == 0)
        def _():
            v1_ref[:, :] = jnp.dot(w1_ref[:, :], wa2_ref[:, :],
                                   preferred_element_type=_F32, precision=_HI)
            v2_ref[:, :] = jnp.dot(w2_ref[:, :], wa2_ref[:, :],
                                   preferred_element_type=_F32, precision=_HI)
            vb_ref[:, :] = jnp.dot(bm_ref[:, :], wa2_ref[:, :],
                                   preferred_element_type=_F32, precision=_HI)

    return pl.pallas_call(
        body,
        grid=(N // BM,),
        in_specs=[
            pl.BlockSpec((BM, D), lambda i: (i, 0)),
            pl.BlockSpec((D, D), lambda i: (0, 0)),
            pl.BlockSpec((D, D), lambda i: (0, 0)),
            pl.BlockSpec((DE, D), lambda i: (0, 0)),
            pl.BlockSpec((D, D), lambda i: (0, 0)),
            pl.BlockSpec((1, D), lambda i: (0, 0)),
            pl.BlockSpec((1, D), lambda i: (0, 0)),
        ],
        out_specs=[
            pl.BlockSpec((BM, D), lambda i: (i, 0)),
            pl.BlockSpec((D, D), lambda i: (0, 0)),
            pl.BlockSpec((DE, D), lambda i: (0, 0)),
            pl.BlockSpec((1, D), lambda i: (0, 0)),
        ],
        out_shape=[
            jax.ShapeDtypeStruct((N, D), _F32),
            jax.ShapeDtypeStruct((D, D), _F32),
            jax.ShapeDtypeStruct((DE, D), _F32),
            jax.ShapeDtypeStruct((1, D), _F32),
        ],
    )(nfeats, wa1, w1, w2, wa2, bm, ba)


def _tc_post(p, sn_p, sd_p, v1, v2, vb):
    BM = 1000

    def body(p_ref, sn_ref, sd_ref, v1_ref, v2_ref, vb_ref, out_ref):
        sd = sd_ref[:, :].astype(_F32)
        deg = sd[:, DE:DE + 1]
        inv = jnp.where(deg > 0.0, 1.0 / jnp.maximum(deg, 1.0), 0.0)
        msk = jnp.where(deg > 0.0, 1.0, 0.0)
        acc = (
            p_ref[:, :]
            + jnp.dot(sn_ref[:, :] * inv, v1_ref[:, :],
                      preferred_element_type=_F32, precision=_HI)
            + jnp.dot(sd[:, 0:DE] * inv, v2_ref[:, :],
                      preferred_element_type=_F32, precision=_HI)
            + msk * vb_ref[:, :]
        )
        out_ref[:, :] = jnp.maximum(acc, 0.0)

    return pl.pallas_call(
        body,
        grid=(N // BM,),
        in_specs=[
            pl.BlockSpec((BM, D), lambda i: (i, 0)),
            pl.BlockSpec((BM, D), lambda i: (i, 0)),
            pl.BlockSpec((BM, 2 * DE), lambda i: (i, 0)),
            pl.BlockSpec((D, D), lambda i: (0, 0)),
            pl.BlockSpec((DE, D), lambda i: (0, 0)),
            pl.BlockSpec((1, D), lambda i: (0, 0)),
        ],
        out_specs=pl.BlockSpec((BM, D), lambda i: (i, 0)),
        out_shape=jax.ShapeDtypeStruct((N, D), _F32),
    )(p, sn_p, sd_p, v1, v2, vb)


def kernel(nfeats, edge_index, efeats, W_msg, b_msg, W_apply, b_apply):
    srci = edge_index[0]
    dsti = edge_index[1]
    sd_pack = jnp.stack(
        [srci.reshape(E // CHUNK, CHUNK),
         dsti.reshape(E // CHUNK, CHUNK)], axis=1).reshape(-1)
    ef_aug = jnp.concatenate(
        [efeats.astype(jnp.bfloat16),
         jnp.ones((E, DE), jnp.bfloat16)], axis=1)
    w1 = W_msg[:D]
    w2 = W_msg[D:]
    wa1 = W_apply[:D]
    wa2 = W_apply[D:]
    p, v1, v2, vb = _tc_pre(nfeats, wa1, w1, w2, wa2,
                            b_msg.reshape(1, D), b_apply.reshape(1, D))
    sn_p, sd_p = _sc_segment_sums(nfeats, sd_pack, ef_aug)
    return _tc_post(p, sn_p, sd_p, v1, v2, vb)
